# baseline (speedup 1.0000x reference)
import functools

import jax
import jax.numpy as jnp
from jax import lax
from jax.experimental import pallas as pl
from jax.experimental.pallas import tpu as pltpu
from jax.experimental.pallas import tpu_sc as plsc

N = 10000
E = 320000
D = 128
H = 128
B = 64

NC = 2
NS = 16
NW = NC * NS
EPW = E // NW
CHUNK = 50
NCHUNK = EPW // CHUNK
ROWS_PT = N // NS


def _sc_mesh():
    return plsc.VectorSubcoreMesh(
        core_axis_name="c", subcore_axis_name="s",
        num_cores=NC, num_subcores=NS)


@functools.partial(
    pl.kernel,
    out_type=jax.ShapeDtypeStruct((NW, N), jnp.float32),
    mesh=_sc_mesh(),
    scratch_types=[
        pltpu.VMEM((EPW,), jnp.int32),
        pltpu.VMEM((N,), jnp.float32),
    ],
    compiler_params=pltpu.CompilerParams(needs_layout_passes=False),
)
def _deg(dst_hbm, out_hbm, dst_v, hist_v):
    cid = lax.axis_index("c")
    sid = lax.axis_index("s")
    wid = cid * NS + sid
    pltpu.sync_copy(dst_hbm.at[wid], dst_v)

    zeros = jnp.zeros((16,), jnp.float32)

    def zbody(i, carry):
        hist_v[pl.ds(i * 16, 16)] = zeros
        return carry

    lax.fori_loop(0, N // 16, zbody, 0, unroll=8)

    ones = jnp.ones((16,), jnp.float32)

    def abody(i, carry):
        idx = dst_v[pl.ds(i * 16, 16)]
        plsc.addupdate_scatter(hist_v, [idx], ones)
        return carry

    lax.fori_loop(0, EPW // 16, abody, 0, unroll=8)
    pltpu.sync_copy(hist_v, out_hbm.at[wid])


@functools.partial(
    pl.kernel,
    out_type=jax.ShapeDtypeStruct((NC, N, H), jnp.float32),
    mesh=_sc_mesh(),
    scratch_types=[
        pltpu.VMEM((NCHUNK, CHUNK), jnp.int32),
        pltpu.VMEM((NCHUNK, CHUNK), jnp.int32),
        pltpu.VMEM((CHUNK, H), jnp.float32),
        pltpu.VMEM((CHUNK, H), jnp.float32),
        pltpu.VMEM_SHARED((N, H), jnp.float32),
        pltpu.SemaphoreType.DMA,
        pltpu.SemaphoreType.DMA,
    ],
    compiler_params=pltpu.CompilerParams(
        needs_layout_passes=False, use_tc_tiling_on_sc=False),
)
def _scatter(g_hbm, src_hbm, dst_hbm, out_hbm,
             src_v, dst_v, buf0, buf1, acc, gsem0, gsem1):
    cid = lax.axis_index("c")
    sid = lax.axis_index("s")
    wid = cid * NS + sid
    base = sid * ROWS_PT

    init = pltpu.async_copy(
        g_hbm.at[pl.ds(base, ROWS_PT)], acc.at[pl.ds(base, ROWS_PT)], gsem0)
    pltpu.sync_copy(src_hbm.at[wid], src_v)
    pltpu.sync_copy(dst_hbm.at[wid], dst_v)
    init.wait()
    plsc.subcore_barrier()

    def gather(j, buf, sem):
        pltpu.async_copy(g_hbm.at[src_v.at[j]], buf, sem)

    def gwait(buf, sem):
        pltpu.make_async_copy(g_hbm.at[src_v.at[0]], buf, sem).wait()

    gather(0, buf0, gsem0)

    def ebody(t, carry):
        j = 2 * t
        gather(j + 1, buf1, gsem1)
        gwait(buf0, gsem0)
        pltpu.sync_copy(buf0, acc.at[dst_v.at[j]], add=True)

        @pl.when(t < NCHUNK // 2 - 1)
        def _():
            gather(j + 2, buf0, gsem0)

        gwait(buf1, gsem1)
        pltpu.sync_copy(buf1, acc.at[dst_v.at[j + 1]], add=True)
        return carry

    lax.fori_loop(0, NCHUNK // 2, ebody, 0)
    plsc.subcore_barrier()
    pltpu.sync_copy(acc.at[pl.ds(base, ROWS_PT)],
                    out_hbm.at[cid, pl.ds(base, ROWS_PT)])


def _prep_body(x_ref, w1_ref, hist_ref, g_ref, dinv_ref):
    deg = jnp.sum(hist_ref[...], axis=0) + 1.0
    dinv = lax.rsqrt(deg)[:, None]
    h = jnp.dot(x_ref[...], w1_ref[...], preferred_element_type=jnp.float32)
    g_ref[...] = h * dinv
    dinv_ref[...] = dinv


def _mid_body(acc_ref, g1_ref, dinv_ref, b1_ref, w2_ref, g2_ref):
    dinv = dinv_ref[...]
    s = acc_ref[0] + acc_ref[1] - g1_ref[...]
    out1 = jnp.maximum(s * dinv + b1_ref[...], 0.0)
    h2 = jnp.dot(out1, w2_ref[...], preferred_element_type=jnp.float32)
    g2_ref[...] = h2 * dinv


def _final_body(acc_ref, g2_ref, dinv_ref, b2_ref, batch_ref, out_ref):
    dinv = dinv_ref[...]
    s = acc_ref[0] + acc_ref[1] - g2_ref[...]
    out2 = jnp.maximum(s * dinv + b2_ref[...], 0.0)
    seg = lax.broadcasted_iota(jnp.int32, (B, N), 0)
    mask = (batch_ref[...] == seg).astype(jnp.float32)
    sums = jnp.dot(mask, out2, preferred_element_type=jnp.float32)
    counts = jnp.sum(mask, axis=1, keepdims=True)
    out_ref[...] = sums / jnp.maximum(counts, 1.0)


_prep = pl.pallas_call(
    _prep_body,
    out_shape=(jax.ShapeDtypeStruct((N, H), jnp.float32),
               jax.ShapeDtypeStruct((N, 1), jnp.float32)),
)

_mid = pl.pallas_call(
    _mid_body,
    out_shape=jax.ShapeDtypeStruct((N, H), jnp.float32),
)

_final = pl.pallas_call(
    _final_body,
    out_shape=jax.ShapeDtypeStruct((B, H), jnp.float32),
)


def kernel(x, edge_index, batch, W1, b1, W2, b2):
    src3 = edge_index[0].reshape(NW, NCHUNK, CHUNK)
    dst3 = edge_index[1].reshape(NW, NCHUNK, CHUNK)
    dst2 = edge_index[1].reshape(NW, EPW)

    hist = _deg(dst2)
    g1, dinv = _prep(x, W1, hist)
    acc1 = _scatter(g1, src3, dst3)
    g2 = _mid(acc1, g1, dinv, b1.reshape(1, H), W2)
    acc2 = _scatter(g2, src3, dst3)
    return _final(acc2, g2, dinv, b2.reshape(1, H), batch.reshape(1, N))


# speedup vs baseline: 1.2733x; 1.2733x over previous
import functools

import jax
import jax.numpy as jnp
from jax import lax
from jax.experimental import pallas as pl
from jax.experimental.pallas import tpu as pltpu
from jax.experimental.pallas import tpu_sc as plsc

N = 10000
E = 320000
D = 128
H = 128
B = 64

NC = 2
NS = 16
NW = NC * NS
EPW = E // NW
CHUNK = 112
NCHUNK = 90
NPADE = NCHUNK * CHUNK - EPW
NP = 10016
ROWS_PT = NP // NS


def _sc_mesh():
    return plsc.VectorSubcoreMesh(
        core_axis_name="c", subcore_axis_name="s",
        num_cores=NC, num_subcores=NS)


@functools.partial(
    pl.kernel,
    out_type=jax.ShapeDtypeStruct((NW, N), jnp.float32),
    mesh=_sc_mesh(),
    scratch_types=[
        pltpu.VMEM((EPW,), jnp.int32),
        pltpu.VMEM((N,), jnp.float32),
    ],
    compiler_params=pltpu.CompilerParams(needs_layout_passes=False),
)
def _deg(dst_hbm, out_hbm, dst_v, hist_v):
    cid = lax.axis_index("c")
    sid = lax.axis_index("s")
    wid = cid * NS + sid
    pltpu.sync_copy(dst_hbm.at[wid], dst_v)

    zeros = jnp.zeros((16,), jnp.float32)

    def zbody(i, carry):
        hist_v[pl.ds(i * 16, 16)] = zeros
        return carry

    lax.fori_loop(0, N // 16, zbody, 0, unroll=8)

    ones = jnp.ones((16,), jnp.float32)

    def abody(i, carry):
        idx = dst_v[pl.ds(i * 16, 16)]
        plsc.addupdate_scatter(hist_v, [idx], ones)
        return carry

    lax.fori_loop(0, EPW // 16, abody, 0, unroll=8)
    pltpu.sync_copy(hist_v, out_hbm.at[wid])


@functools.partial(
    pl.kernel,
    out_type=jax.ShapeDtypeStruct((NC, NP, H), jnp.float32),
    mesh=_sc_mesh(),
    scratch_types=[
        pltpu.VMEM((NCHUNK, CHUNK), jnp.int32),
        pltpu.VMEM((NCHUNK, CHUNK), jnp.int32),
        pltpu.VMEM((CHUNK, H), jnp.float32),
        pltpu.VMEM((CHUNK, H), jnp.float32),
        pltpu.VMEM_SHARED((NP, H), jnp.float32),
        pltpu.SemaphoreType.DMA,
        pltpu.SemaphoreType.DMA,
    ],
    compiler_params=pltpu.CompilerParams(
        needs_layout_passes=False, use_tc_tiling_on_sc=False),
)
def _scatter(g_hbm, src_hbm, dst_hbm, out_hbm,
             src_v, dst_v, buf0, buf1, acc, gsem0, gsem1):
    cid = lax.axis_index("c")
    sid = lax.axis_index("s")
    wid = cid * NS + sid
    base = sid * ROWS_PT

    init = pltpu.async_copy(
        g_hbm.at[pl.ds(base, ROWS_PT)], acc.at[pl.ds(base, ROWS_PT)], gsem0)
    pltpu.sync_copy(src_hbm.at[wid], src_v)
    pltpu.sync_copy(dst_hbm.at[wid], dst_v)
    init.wait()
    plsc.subcore_barrier()

    def gather(j, buf, sem):
        pltpu.async_copy(g_hbm.at[src_v.at[j]], buf, sem)

    def gwait(buf, sem):
        pltpu.make_async_copy(g_hbm.at[src_v.at[0]], buf, sem).wait()

    gather(0, buf0, gsem0)

    def ebody(t, carry):
        j = 2 * t
        gather(j + 1, buf1, gsem1)
        gwait(buf0, gsem0)
        pltpu.sync_copy(buf0, acc.at[dst_v.at[j]], add=True)

        @pl.when(t < NCHUNK // 2 - 1)
        def _():
            gather(j + 2, buf0, gsem0)

        gwait(buf1, gsem1)
        pltpu.sync_copy(buf1, acc.at[dst_v.at[j + 1]], add=True)
        return carry

    lax.fori_loop(0, NCHUNK // 2, ebody, 0)
    plsc.subcore_barrier()
    pltpu.sync_copy(acc.at[pl.ds(base, ROWS_PT)],
                    out_hbm.at[cid, pl.ds(base, ROWS_PT)])


def _prep_body(x_ref, w1_ref, hist_ref, g_ref, dinv_ref):
    deg = jnp.sum(hist_ref[...], axis=0) + 1.0
    dinv = lax.rsqrt(deg)[:, None]
    h = jnp.dot(x_ref[...], w1_ref[...], preferred_element_type=jnp.float32)
    g_ref[0:N, :] = h * dinv
    g_ref[N:NP, :] = jnp.zeros((NP - N, H), jnp.float32)
    dinv_ref[...] = dinv


def _mid_body(acc_ref, g1_ref, dinv_ref, b1_ref, w2_ref, g2_ref):
    dinv = dinv_ref[...]
    s = acc_ref[0, 0:N, :] + acc_ref[1, 0:N, :] - g1_ref[0:N, :]
    out1 = jnp.maximum(s * dinv + b1_ref[...], 0.0)
    h2 = jnp.dot(out1, w2_ref[...], preferred_element_type=jnp.float32)
    g2_ref[0:N, :] = h2 * dinv
    g2_ref[N:NP, :] = jnp.zeros((NP - N, H), jnp.float32)


def _final_body(acc_ref, g2_ref, dinv_ref, b2_ref, batch_ref, out_ref):
    dinv = dinv_ref[...]
    s = acc_ref[0, 0:N, :] + acc_ref[1, 0:N, :] - g2_ref[0:N, :]
    out2 = jnp.maximum(s * dinv + b2_ref[...], 0.0)
    seg = lax.broadcasted_iota(jnp.int32, (B, N), 0)
    mask = (batch_ref[...] == seg).astype(jnp.float32)
    sums = jnp.dot(mask, out2, preferred_element_type=jnp.float32)
    counts = jnp.sum(mask, axis=1, keepdims=True)
    out_ref[...] = sums / jnp.maximum(counts, 1.0)


_prep = pl.pallas_call(
    _prep_body,
    out_shape=(jax.ShapeDtypeStruct((NP, H), jnp.float32),
               jax.ShapeDtypeStruct((N, 1), jnp.float32)),
)

_mid = pl.pallas_call(
    _mid_body,
    out_shape=jax.ShapeDtypeStruct((NP, H), jnp.float32),
)

_final = pl.pallas_call(
    _final_body,
    out_shape=jax.ShapeDtypeStruct((B, H), jnp.float32),
)


def kernel(x, edge_index, batch, W1, b1, W2, b2):
    pad = jnp.broadcast_to(N + (jnp.arange(NPADE, dtype=jnp.int32) % (NP - N)),
                           (NW, NPADE))
    src3 = jnp.concatenate(
        [edge_index[0].reshape(NW, EPW), pad], axis=1).reshape(NW, NCHUNK, CHUNK)
    dst3 = jnp.concatenate(
        [edge_index[1].reshape(NW, EPW), pad], axis=1).reshape(NW, NCHUNK, CHUNK)
    dst2 = edge_index[1].reshape(NW, EPW)

    hist = _deg(dst2)
    g1, dinv = _prep(x, W1, hist)
    acc1 = _scatter(g1, src3, dst3)
    g2 = _mid(acc1, g1, dinv, b1.reshape(1, H), W2)
    acc2 = _scatter(g2, src3, dst3)
    return _final(acc2, g2, dinv, b2.reshape(1, H), batch.reshape(1, N))


# speedup vs baseline: 1.3904x; 1.0920x over previous
import functools

import jax
import jax.numpy as jnp
from jax import lax
from jax.experimental import pallas as pl
from jax.experimental.pallas import tpu as pltpu
from jax.experimental.pallas import tpu_sc as plsc

N = 10000
E = 320000
D = 128
H = 128
B = 64

NC = 2
NS = 16
NW = NC * NS
EPW = E // NW
CHUNK = 80
NCHUNK = EPW // CHUNK
ROWS_PT = N // NS


def _sc_mesh():
    return plsc.VectorSubcoreMesh(
        core_axis_name="c", subcore_axis_name="s",
        num_cores=NC, num_subcores=NS)


@functools.partial(
    pl.kernel,
    out_type=jax.ShapeDtypeStruct((NW, N), jnp.float32),
    mesh=_sc_mesh(),
    scratch_types=[
        pltpu.VMEM((EPW,), jnp.int32),
        pltpu.VMEM((N,), jnp.float32),
    ],
    compiler_params=pltpu.CompilerParams(needs_layout_passes=False),
)
def _deg(dst_hbm, out_hbm, dst_v, hist_v):
    cid = lax.axis_index("c")
    sid = lax.axis_index("s")
    wid = cid * NS + sid
    pltpu.sync_copy(dst_hbm.at[wid], dst_v)

    zeros = jnp.zeros((16,), jnp.float32)

    def zbody(i, carry):
        hist_v[pl.ds(i * 16, 16)] = zeros
        return carry

    lax.fori_loop(0, N // 16, zbody, 0, unroll=8)

    ones = jnp.ones((16,), jnp.float32)

    def abody(i, carry):
        idx = dst_v[pl.ds(i * 16, 16)]
        plsc.addupdate_scatter(hist_v, [idx], ones)
        return carry

    lax.fori_loop(0, EPW // 16, abody, 0, unroll=8)
    pltpu.sync_copy(hist_v, out_hbm.at[wid])


@functools.partial(
    pl.kernel,
    out_type=jax.ShapeDtypeStruct((NC, N, H), jnp.float32),
    mesh=_sc_mesh(),
    scratch_types=[
        pltpu.VMEM((NCHUNK, CHUNK), jnp.int32),
        pltpu.VMEM((NCHUNK, CHUNK), jnp.int32),
        pltpu.VMEM((CHUNK, H), jnp.float32),
        pltpu.VMEM((CHUNK, H), jnp.float32),
        pltpu.VMEM((CHUNK, H), jnp.float32),
        pltpu.VMEM_SHARED((N, H), jnp.float32),
        pltpu.SemaphoreType.DMA,
        pltpu.SemaphoreType.DMA,
        pltpu.SemaphoreType.DMA,
        pltpu.SemaphoreType.DMA,
        pltpu.SemaphoreType.DMA,
        pltpu.SemaphoreType.DMA,
    ],
    compiler_params=pltpu.CompilerParams(
        needs_layout_passes=False, use_tc_tiling_on_sc=False),
)
def _scatter(g_hbm, src_hbm, dst_hbm, out_hbm,
             src_v, dst_v, buf0, buf1, buf2, acc,
             gsem0, gsem1, gsem2, ssem0, ssem1, ssem2):
    cid = lax.axis_index("c")
    sid = lax.axis_index("s")
    wid = cid * NS + sid
    base = sid * ROWS_PT
    bufs = (buf0, buf1, buf2)
    gsems = (gsem0, gsem1, gsem2)
    ssems = (ssem0, ssem1, ssem2)

    init = pltpu.async_copy(
        g_hbm.at[pl.ds(base, ROWS_PT)], acc.at[pl.ds(base, ROWS_PT)], gsem0)
    pltpu.sync_copy(src_hbm.at[wid], src_v)
    pltpu.sync_copy(dst_hbm.at[wid], dst_v)
    init.wait()
    plsc.subcore_barrier()

    def gissue(j, k):
        pltpu.async_copy(g_hbm.at[src_v.at[j]], bufs[k], gsems[k])

    def gwait(k):
        pltpu.make_async_copy(g_hbm.at[src_v.at[0]], bufs[k], gsems[k]).wait()

    def sissue(j, k):
        pltpu.async_copy(bufs[k], acc.at[dst_v.at[j]], ssems[k], add=True)

    def swait(k):
        pltpu.make_async_copy(bufs[k], acc.at[dst_v.at[0]], ssems[k]).wait()

    gissue(0, 0)
    gissue(1, 1)

    def ebody(t, carry):
        for k in range(3):
            j = 3 * t + k
            gwait(k)
            sissue(j, k)
            if k == 0:
                @pl.when(t > 0)
                def _():
                    swait(2)
            else:
                swait(k - 1)
            gissue(j + 2, (k + 2) % 3)
        return carry

    lax.fori_loop(0, (NCHUNK - 2) // 3, ebody, 0)
    gwait(0)
    swait(2)
    sissue(NCHUNK - 2, 0)
    gwait(1)
    swait(0)
    sissue(NCHUNK - 1, 1)
    swait(1)
    plsc.subcore_barrier()
    pltpu.sync_copy(acc.at[pl.ds(base, ROWS_PT)],
                    out_hbm.at[cid, pl.ds(base, ROWS_PT)])


def _prep_body(x_ref, w1_ref, hist_ref, g_ref, dinv_ref):
    deg = jnp.sum(hist_ref[...], axis=0) + 1.0
    dinv = lax.rsqrt(deg)[:, None]
    h = jnp.dot(x_ref[...], w1_ref[...], preferred_element_type=jnp.float32)
    g_ref[...] = h * dinv
    dinv_ref[...] = dinv


def _mid_body(acc_ref, g1_ref, dinv_ref, b1_ref, w2_ref, g2_ref):
    dinv = dinv_ref[...]
    s = acc_ref[0] + acc_ref[1] - g1_ref[...]
    out1 = jnp.maximum(s * dinv + b1_ref[...], 0.0)
    h2 = jnp.dot(out1, w2_ref[...], preferred_element_type=jnp.float32)
    g2_ref[...] = h2 * dinv


def _final_body(acc_ref, g2_ref, dinv_ref, b2_ref, batch_ref, out_ref):
    dinv = dinv_ref[...]
    s = acc_ref[0] + acc_ref[1] - g2_ref[...]
    out2 = jnp.maximum(s * dinv + b2_ref[...], 0.0)
    seg = lax.broadcasted_iota(jnp.int32, (B, N), 0)
    mask = (batch_ref[...] == seg).astype(jnp.float32)
    sums = jnp.dot(mask, out2, preferred_element_type=jnp.float32)
    counts = jnp.sum(mask, axis=1, keepdims=True)
    out_ref[...] = sums / jnp.maximum(counts, 1.0)


_prep = pl.pallas_call(
    _prep_body,
    out_shape=(jax.ShapeDtypeStruct((N, H), jnp.float32),
               jax.ShapeDtypeStruct((N, 1), jnp.float32)),
)

_mid = pl.pallas_call(
    _mid_body,
    out_shape=jax.ShapeDtypeStruct((N, H), jnp.float32),
)

_final = pl.pallas_call(
    _final_body,
    out_shape=jax.ShapeDtypeStruct((B, H), jnp.float32),
)


def kernel(x, edge_index, batch, W1, b1, W2, b2):
    src3 = edge_index[0].reshape(NW, NCHUNK, CHUNK)
    dst3 = edge_index[1].reshape(NW, NCHUNK, CHUNK)
    dst2 = edge_index[1].reshape(NW, EPW)

    hist = _deg(dst2)
    g1, dinv = _prep(x, W1, hist)
    acc1 = _scatter(g1, src3, dst3)
    g2 = _mid(acc1, g1, dinv, b1.reshape(1, H), W2)
    acc2 = _scatter(g2, src3, dst3)
    return _final(acc2, g2, dinv, b2.reshape(1, H), batch.reshape(1, N))


# speedup vs baseline: 1.4032x; 1.0092x over previous
import functools

import jax
import jax.numpy as jnp
from jax import lax
from jax.experimental import pallas as pl
from jax.experimental.pallas import tpu as pltpu
from jax.experimental.pallas import tpu_sc as plsc

N = 10000
E = 320000
D = 128
H = 128
B = 64

NC = 2
NS = 16
NW = NC * NS
EPW = E // NW
CHUNK = 80
NCHUNK = EPW // CHUNK
ROWS_PT = N // NS


def _sc_mesh():
    return plsc.VectorSubcoreMesh(
        core_axis_name="c", subcore_axis_name="s",
        num_cores=NC, num_subcores=NS)


@functools.partial(
    pl.kernel,
    out_type=jax.ShapeDtypeStruct((NW, N), jnp.float32),
    mesh=_sc_mesh(),
    scratch_types=[
        pltpu.VMEM((EPW,), jnp.int32),
        pltpu.VMEM((N,), jnp.float32),
    ],
    compiler_params=pltpu.CompilerParams(needs_layout_passes=False),
)
def _deg(dst_hbm, out_hbm, dst_v, hist_v):
    cid = lax.axis_index("c")
    sid = lax.axis_index("s")
    wid = cid * NS + sid
    pltpu.sync_copy(dst_hbm.at[wid], dst_v)

    zeros = jnp.zeros((16,), jnp.float32)

    def zbody(i, carry):
        hist_v[pl.ds(i * 16, 16)] = zeros
        return carry

    lax.fori_loop(0, N // 16, zbody, 0, unroll=8)

    ones = jnp.ones((16,), jnp.float32)

    def abody(i, carry):
        idx = dst_v[pl.ds(i * 16, 16)]
        plsc.addupdate_scatter(hist_v, [idx], ones)
        return carry

    lax.fori_loop(0, EPW // 16, abody, 0, unroll=8)
    pltpu.sync_copy(hist_v, out_hbm.at[wid])


@functools.partial(
    pl.kernel,
    out_type=jax.ShapeDtypeStruct((NC, N, H), jnp.bfloat16),
    mesh=_sc_mesh(),
    scratch_types=[
        pltpu.VMEM((NCHUNK, CHUNK), jnp.int32),
        pltpu.VMEM((NCHUNK, CHUNK), jnp.int32),
        pltpu.VMEM((CHUNK, H), jnp.bfloat16),
        pltpu.VMEM((CHUNK, H), jnp.bfloat16),
        pltpu.VMEM((CHUNK, H), jnp.bfloat16),
        pltpu.VMEM_SHARED((N, H), jnp.bfloat16),
        pltpu.SemaphoreType.DMA,
        pltpu.SemaphoreType.DMA,
        pltpu.SemaphoreType.DMA,
        pltpu.SemaphoreType.DMA,
        pltpu.SemaphoreType.DMA,
        pltpu.SemaphoreType.DMA,
    ],
    compiler_params=pltpu.CompilerParams(
        needs_layout_passes=False, use_tc_tiling_on_sc=False),
)
def _scatter(g_hbm, src_hbm, dst_hbm, out_hbm,
             src_v, dst_v, buf0, buf1, buf2, acc,
             gsem0, gsem1, gsem2, ssem0, ssem1, ssem2):
    cid = lax.axis_index("c")
    sid = lax.axis_index("s")
    wid = cid * NS + sid
    base = sid * ROWS_PT
    bufs = (buf0, buf1, buf2)
    gsems = (gsem0, gsem1, gsem2)
    ssems = (ssem0, ssem1, ssem2)

    init = pltpu.async_copy(
        g_hbm.at[pl.ds(base, ROWS_PT)], acc.at[pl.ds(base, ROWS_PT)], gsem0)
    pltpu.sync_copy(src_hbm.at[wid], src_v)
    pltpu.sync_copy(dst_hbm.at[wid], dst_v)
    init.wait()
    plsc.subcore_barrier()

    def gissue(j, k):
        pltpu.async_copy(g_hbm.at[src_v.at[j]], bufs[k], gsems[k])

    def gwait(k):
        pltpu.make_async_copy(g_hbm.at[src_v.at[0]], bufs[k], gsems[k]).wait()

    def sissue(j, k):
        pltpu.async_copy(bufs[k], acc.at[dst_v.at[j]], ssems[k], add=True)

    def swait(k):
        pltpu.make_async_copy(bufs[k], acc.at[dst_v.at[0]], ssems[k]).wait()

    gissue(0, 0)
    gissue(1, 1)

    def ebody(t, carry):
        for k in range(3):
            j = 3 * t + k
            gwait(k)
            sissue(j, k)
            if k == 0:
                @pl.when(t > 0)
                def _():
                    swait(2)
            else:
                swait(k - 1)
            gissue(j + 2, (k + 2) % 3)
        return carry

    lax.fori_loop(0, (NCHUNK - 2) // 3, ebody, 0)
    gwait(0)
    swait(2)
    sissue(NCHUNK - 2, 0)
    gwait(1)
    swait(0)
    sissue(NCHUNK - 1, 1)
    swait(1)
    plsc.subcore_barrier()
    pltpu.sync_copy(acc.at[pl.ds(base, ROWS_PT)],
                    out_hbm.at[cid, pl.ds(base, ROWS_PT)])


def _prep_body(x_ref, w1_ref, hist_ref, g_ref, dinv_ref):
    deg = jnp.sum(hist_ref[...], axis=0) + 1.0
    dinv = lax.rsqrt(deg)[:, None]
    h = jnp.dot(x_ref[...], w1_ref[...], preferred_element_type=jnp.float32)
    g_ref[...] = (h * dinv).astype(jnp.bfloat16)
    dinv_ref[...] = dinv


def _mid_body(acc_ref, g1_ref, dinv_ref, b1_ref, w2_ref, g2_ref):
    dinv = dinv_ref[...]
    s = (acc_ref[0].astype(jnp.float32) + acc_ref[1].astype(jnp.float32)
         - g1_ref[...].astype(jnp.float32))
    out1 = jnp.maximum(s * dinv + b1_ref[...], 0.0)
    h2 = jnp.dot(out1, w2_ref[...], preferred_element_type=jnp.float32)
    g2_ref[...] = (h2 * dinv).astype(jnp.bfloat16)


def _final_body(acc_ref, g2_ref, dinv_ref, b2_ref, batch_ref, out_ref):
    dinv = dinv_ref[...]
    s = (acc_ref[0].astype(jnp.float32) + acc_ref[1].astype(jnp.float32)
         - g2_ref[...].astype(jnp.float32))
    out2 = jnp.maximum(s * dinv + b2_ref[...], 0.0)
    seg = lax.broadcasted_iota(jnp.int32, (B, N), 0)
    mask = (batch_ref[...] == seg).astype(jnp.float32)
    sums = jnp.dot(mask, out2, preferred_element_type=jnp.float32)
    counts = jnp.sum(mask, axis=1, keepdims=True)
    out_ref[...] = sums / jnp.maximum(counts, 1.0)


_prep = pl.pallas_call(
    _prep_body,
    out_shape=(jax.ShapeDtypeStruct((N, H), jnp.bfloat16),
               jax.ShapeDtypeStruct((N, 1), jnp.float32)),
)

_mid = pl.pallas_call(
    _mid_body,
    out_shape=jax.ShapeDtypeStruct((N, H), jnp.bfloat16),
)

_final = pl.pallas_call(
    _final_body,
    out_shape=jax.ShapeDtypeStruct((B, H), jnp.float32),
)


def kernel(x, edge_index, batch, W1, b1, W2, b2):
    src3 = edge_index[0].reshape(NW, NCHUNK, CHUNK)
    dst3 = edge_index[1].reshape(NW, NCHUNK, CHUNK)
    dst2 = edge_index[1].reshape(NW, EPW)

    hist = _deg(dst2)
    g1, dinv = _prep(x, W1, hist)
    acc1 = _scatter(g1, src3, dst3)
    g2 = _mid(acc1, g1, dinv, b1.reshape(1, H), W2)
    acc2 = _scatter(g2, src3, dst3)
    return _final(acc2, g2, dinv, b2.reshape(1, H), batch.reshape(1, N))
